# baseline (speedup 1.0000x reference)
import jax
import jax.numpy as jnp
from jax.experimental import pallas as pl
from jax.experimental.pallas import tpu as pltpu

_EOS = 2
_CR = 8


def _eos_kernel(ids_ref, scores_alias, out_hbm, done_ref, buf_ref, sem):
    rows = ids_ref.shape[0]
    counts = jnp.sum((ids_ref[...] == _EOS).astype(jnp.int32), axis=1,
                     keepdims=True)
    count_init = counts
    done = (counts - count_init) >= 2
    done_ref[...] = done.astype(jnp.float32)
    n_done = jnp.sum(done.astype(jnp.int32))

    @pl.when(n_done != 0)
    def _masked():
        for c in range(rows // _CR):
            cp_in = pltpu.make_async_copy(
                out_hbm.at[pl.ds(c * _CR, _CR), :], buf_ref, sem)
            cp_in.start()
            cp_in.wait()
            done_c = done_ref[pl.ds(c * _CR, _CR), :] > 0.0
            block = buf_ref[...]
            masked = jnp.where(done_c, -jnp.inf, block)
            buf_ref[...] = masked
            buf_ref[:, _EOS:_EOS + 1] = jnp.where(
                done_c, 0.0, block[:, _EOS:_EOS + 1])
            cp_out = pltpu.make_async_copy(
                buf_ref, out_hbm.at[pl.ds(c * _CR, _CR), :], sem)
            cp_out.start()
            cp_out.wait()


def kernel(input_ids, scores):
    batch, vocab = scores.shape
    zero = jnp.minimum(input_ids[0, 0], 0).astype(scores.dtype)
    scores = scores + zero
    return pl.pallas_call(
        _eos_kernel,
        in_specs=[
            pl.BlockSpec(input_ids.shape, lambda: (0, 0)),
            pl.BlockSpec(memory_space=pl.ANY),
        ],
        out_specs=pl.BlockSpec(memory_space=pl.ANY),
        out_shape=jax.ShapeDtypeStruct(scores.shape, scores.dtype),
        input_output_aliases={1: 0},
        scratch_shapes=[
            pltpu.VMEM((batch, 1), jnp.float32),
            pltpu.VMEM((_CR, vocab), jnp.float32),
            pltpu.SemaphoreType.DMA,
        ],
    )(input_ids, scores)


# speedup vs baseline: 1.4604x; 1.4604x over previous
import jax
import jax.numpy as jnp
from jax.experimental import pallas as pl
from jax.experimental.pallas import tpu as pltpu

_EOS = 2
_CR = 8


def _eos_kernel(ids_ref, scores_alias, out_hbm, done_ref, buf_ref, sem):
    rows = ids_ref.shape[0]
    counts = jnp.sum((ids_ref[...] == _EOS).astype(jnp.int32), axis=1,
                     keepdims=True)
    count_init = counts
    done = (counts - count_init) >= 2
    done_ref[...] = done.astype(jnp.float32)
    n_done = jnp.sum(done.astype(jnp.int32))

    @pl.when(n_done != 0)
    def _masked():
        for c in range(rows // _CR):
            cp_in = pltpu.make_async_copy(
                out_hbm.at[pl.ds(c * _CR, _CR), :], buf_ref, sem)
            cp_in.start()
            cp_in.wait()
            done_c = done_ref[pl.ds(c * _CR, _CR), :] > 0.0
            block = buf_ref[...]
            masked = jnp.where(done_c, -jnp.inf, block)
            buf_ref[...] = masked
            buf_ref[:, _EOS:_EOS + 1] = jnp.where(
                done_c, 0.0, block[:, _EOS:_EOS + 1])
            cp_out = pltpu.make_async_copy(
                buf_ref, out_hbm.at[pl.ds(c * _CR, _CR), :], sem)
            cp_out.start()
            cp_out.wait()


def kernel(input_ids, scores):
    batch, vocab = scores.shape
    return pl.pallas_call(
        _eos_kernel,
        in_specs=[
            pl.BlockSpec(input_ids.shape, lambda: (0, 0)),
            pl.BlockSpec(memory_space=pl.ANY),
        ],
        out_specs=pl.BlockSpec(memory_space=pl.ANY),
        out_shape=jax.ShapeDtypeStruct(scores.shape, scores.dtype),
        input_output_aliases={1: 0},
        scratch_shapes=[
            pltpu.VMEM((batch, 1), jnp.float32),
            pltpu.VMEM((_CR, vocab), jnp.float32),
            pltpu.SemaphoreType.DMA,
        ],
    )(input_ids, scores)
